# baseline (speedup 1.0000x reference)
import jax
import jax.numpy as jnp
from jax.experimental import pallas as pl
from jax.experimental.pallas import tpu as pltpu

_N = 16384
_ACTION_DIM = 7
_EXPOSURE_THRESHOLD = 0.9
_B = 2048

def _body(pos_ref, port_ref, out_ref):
    ps = pos_ref[...][:, 1:2]
    ex = port_ref[...][:, 1:2]
    has = ps > 0.0
    high = ex >= _EXPOSURE_THRESHOLD
    col = jax.lax.broadcasted_iota(jnp.int32, (ps.shape[0], _ACTION_DIM), 1)
    buy = (col >= 1) & (col <= 3)
    sell = col >= 4
    inc = buy | (col == 6)
    mask = ~(~has & sell) & ~(has & buy) & ~(high & inc)
    out_ref[...] = mask | (col == 0)


def kernel(position, portfolio):
    position = position.astype(jnp.float32)
    portfolio = portfolio.astype(jnp.float32)
    return pl.pallas_call(
        _body,
        grid=(_N // _B,),
        in_specs=[
            pl.BlockSpec((_B, 5), lambda i: (i, 0)),
            pl.BlockSpec((_B, 8), lambda i: (i, 0)),
        ],
        out_specs=pl.BlockSpec((_B, _ACTION_DIM), lambda i: (i, 0)),
        out_shape=jax.ShapeDtypeStruct((_N, _ACTION_DIM), jnp.bool_),
    )(position, portfolio)
